# baseline (speedup 1.0000x reference)
import functools

import jax
import jax.numpy as jnp
from jax import lax
from jax.experimental import pallas as pl
from jax.experimental.pallas import tpu as pltpu
from jax.experimental.pallas import tpu_sc as plsc

N = 10000
E = 320000
D = 128
G = 256

NC = 2
NS = 16
K = 64
NW = NC * NS
EPW = 10240
EPAD = EPW * NW
CH = EPW // K
IB = 16
NCU = 1
CHW = EPAD // K // (NCU * NS)
TPR = 632
NROWS = TPR * NS

RB = 400
NBLK = N // RB


R = 4
F = 2


def _sc_agg_body(table_hbm, src_hbm, dst_hbm, zrows_hbm, out_hbm,
                 sidx, didx, rows, agg_sh, zsem, gsem, ssem, isem):
    c = lax.axis_index("c")
    s = lax.axis_index("s")
    zdesc = pltpu.async_copy(zrows_hbm, agg_sh.at[pl.ds(s * TPR, TPR)], zsem)
    ch_c = CHW
    nib_c = CHW // IB
    base_row = pl.multiple_of((c * NS + s) * CHW, 8)
    pltpu.sync_copy(src_hbm.at[pl.ds(base_row, IB)], sidx.at[0])
    pltpu.sync_copy(dst_hbm.at[pl.ds(base_row, IB)], didx.at[0])
    for t in range(F):
        pltpu.async_copy(table_hbm.at[sidx.at[0, t]], rows.at[t], gsem)
    zdesc.wait()
    plsc.subcore_barrier()

    def drain(sem):
        pltpu.make_async_copy(table_hbm.at[pl.ds(0, K)], rows.at[0],
                              sem).wait()

    def body(t, _):
        blk = t // IB
        r_in = t % IB
        nxt_ok = blk + 1 < nib_c

        @pl.when(jnp.logical_and(r_in == 0, nxt_ok))
        def _():
            nxt = (blk + 1) % 2
            row0 = pl.multiple_of(base_row + (blk + 1) * IB, 8)
            pltpu.async_copy(src_hbm.at[pl.ds(row0, IB)], sidx.at[nxt], isem)
            pltpu.async_copy(dst_hbm.at[pl.ds(row0, IB)], didx.at[nxt], isem)

        @pl.when(jnp.logical_and(r_in == IB - F - 1, nxt_ok))
        def _():
            for ref in (sidx, didx):
                pltpu.make_async_copy(src_hbm.at[pl.ds(base_row, IB)],
                                      ref.at[0], isem).wait()

        drain(gsem)
        cur = blk % 2
        pltpu.async_copy(rows.at[lax.rem(t, R)], agg_sh.at[didx.at[cur, r_in]],
                         ssem, add=True)

        @pl.when(t + F < ch_c)
        def _():
            @pl.when(t >= R - F)
            def _():
                drain(ssem)
            tf = t + F
            blkf = tf // IB
            pltpu.async_copy(table_hbm.at[sidx.at[lax.rem(blkf, 2),
                                                  lax.rem(tf, IB)]],
                             rows.at[lax.rem(tf, R)], gsem)
        return 0

    lax.fori_loop(0, ch_c, body, 0)
    for _ in range(R):
        drain(ssem)
    plsc.subcore_barrier()
    pltpu.sync_copy(agg_sh.at[pl.ds(s * TPR, TPR)],
                    out_hbm.at[c, pl.ds(s * TPR, TPR)])


_sc_agg = functools.partial(
    pl.kernel,
    out_type=jax.ShapeDtypeStruct((NCU, NROWS, D), jnp.float32),
    mesh=plsc.VectorSubcoreMesh(core_axis_name="c", subcore_axis_name="s",
                                num_cores=NCU),
    scratch_types=[
        pltpu.VMEM((2, IB, K), jnp.int32),
        pltpu.VMEM((2, IB, K), jnp.int32),
        pltpu.VMEM((R, K, D), jnp.float32),
        pltpu.VMEM_SHARED((NROWS, D), jnp.float32),
        pltpu.SemaphoreType.DMA,
        pltpu.SemaphoreType.DMA,
        pltpu.SemaphoreType.DMA,
        pltpu.SemaphoreType.DMA,
    ],
)(_sc_agg_body)


def _tc_layer1_body(p_ref, x_ref, wrel_ref, wroot_ref, b_ref, h_ref):
    agg = p_ref[0]
    h = (jnp.dot(agg, wrel_ref[...], preferred_element_type=jnp.float32)
         + jnp.dot(x_ref[...], wroot_ref[...],
                   preferred_element_type=jnp.float32)
         + b_ref[...])
    h_ref[...] = jnp.maximum(h, 0.0)


def _tc_layer1(p, x, wrel, wroot, b2d):
    return pl.pallas_call(
        _tc_layer1_body,
        grid=(NBLK,),
        in_specs=[
            pl.BlockSpec((NCU, RB, D), lambda i: (0, i, 0)),
            pl.BlockSpec((RB, D), lambda i: (i, 0)),
            pl.BlockSpec((D, D), lambda i: (0, 0)),
            pl.BlockSpec((D, D), lambda i: (0, 0)),
            pl.BlockSpec((1, D), lambda i: (0, 0)),
        ],
        out_specs=pl.BlockSpec((RB, D), lambda i: (i, 0)),
        out_shape=jax.ShapeDtypeStruct((N, D), jnp.float32),
    )(p, x, wrel, wroot, b2d)


def _tc_layer2_body(p_ref, h_ref, wrel_ref, wroot_ref, b_ref, batch_ref,
                    out_ref, acc, cnt):
    i = pl.program_id(0)
    agg = p_ref[0]
    h2 = (jnp.dot(agg, wrel_ref[...], preferred_element_type=jnp.float32)
          + jnp.dot(h_ref[...], wroot_ref[...],
                    preferred_element_type=jnp.float32)
          + b_ref[...])
    h2 = jnp.maximum(h2, 0.0)

    seg = batch_ref[0, 0, :]
    gid = lax.broadcasted_iota(jnp.int32, (RB, G), 1)
    onehot = jnp.where(seg[:, None] == gid, 1.0, 0.0)
    psum = lax.dot_general(onehot, h2, (((0,), (0,)), ((), ())),
                           preferred_element_type=jnp.float32)
    pcnt = jnp.sum(onehot, axis=0)[None, :]

    @pl.when(i == 0)
    def _():
        acc[...] = psum
        cnt[...] = pcnt

    @pl.when(i > 0)
    def _():
        acc[...] += psum
        cnt[...] += pcnt

    @pl.when(i == NBLK - 1)
    def _():
        denom = jnp.maximum(cnt[...], 1.0)
        out_ref[...] = acc[...] / denom[0, :, None]


def _tc_layer2(p, h, wrel, wroot, b2d, batch3):
    return pl.pallas_call(
        _tc_layer2_body,
        grid=(NBLK,),
        in_specs=[
            pl.BlockSpec((NCU, RB, D), lambda i: (0, i, 0)),
            pl.BlockSpec((RB, D), lambda i: (i, 0)),
            pl.BlockSpec((D, D), lambda i: (0, 0)),
            pl.BlockSpec((D, D), lambda i: (0, 0)),
            pl.BlockSpec((1, D), lambda i: (0, 0)),
            pl.BlockSpec((1, 1, RB), lambda i: (i, 0, 0)),
        ],
        out_specs=pl.BlockSpec((G, D), lambda i: (0, 0)),
        out_shape=jax.ShapeDtypeStruct((G, D), jnp.float32),
        scratch_shapes=[
            pltpu.VMEM((G, D), jnp.float32),
            pltpu.VMEM((1, G), jnp.float32),
        ],
        compiler_params=pltpu.CompilerParams(
            dimension_semantics=("arbitrary",),
        ),
    )(p, h, wrel, wroot, b2d, batch3)


def kernel(x, edge_index, batch, W_rel1, b_rel1, W_root1, W_rel2, b_rel2,
           W_root2):
    src = edge_index[0]
    dst = edge_index[1]
    npad = EPAD - E
    src_p = jnp.concatenate([src, jnp.zeros((npad,), jnp.int32)])
    src_p = src_p.reshape(EPAD // K, K)
    trash = N + (jnp.arange(npad, dtype=jnp.int32) % (NROWS - N))
    dst_p = jnp.concatenate([dst, trash]).reshape(EPAD // K, K)
    zrows = jnp.zeros((TPR, D), jnp.float32)
    b1 = b_rel1.reshape(1, D)
    b2 = b_rel2.reshape(1, D)
    batch3 = batch.reshape(NBLK, 1, RB)

    p1 = _sc_agg(x, src_p, dst_p, zrows)
    h = _tc_layer1(p1, x, W_rel1, W_root1, b1)
    p2 = _sc_agg(h, src_p, dst_p, zrows)
    out = _tc_layer2(p2, h, W_rel2, W_root2, b2, batch3)
    return out


# speedup vs baseline: 1.6260x; 1.6260x over previous
import functools

import jax
import jax.numpy as jnp
import numpy as np
from jax import lax
from jax.experimental import pallas as pl
from jax.experimental.pallas import tpu as pltpu
from jax.experimental.pallas import tpu_sc as plsc

N = 10000
E = 320000
D = 128
G = 256

NC = 2
NS = 16
NCU = 2
K = 64
EPW = 10240
EPAD = EPW * NC * NS
CHW = EPAD // K // (NCU * NS)
IB = 16
NIBW = CHW // IB
TPR = 632
NROWS = TPR * NS

R = 4
S = 2
F = 3

RB = 400
NBLK = N // RB

_CM = np.zeros(D, np.int32)
for _g in range(4):
    for _j in range(16):
        _CM[32 * _g + 2 * _j] = 32 * _g + _j
        _CM[32 * _g + 2 * _j + 1] = 32 * _g + 16 + _j
_PM = np.zeros((D, D), np.float32)
for _p in range(D):
    _PM[_CM[_p], _p] = 1.0


def _sc_agg_body(table_hbm, src_hbm, dst_hbm, zrows_hbm, out_hbm,
                 sidx, didx, rows_bf, rows_f32, agg_sh,
                 zsem, gsem, ssem, isem):
    c = lax.axis_index("c")
    s = lax.axis_index("s")
    zdesc = pltpu.async_copy(zrows_hbm, agg_sh.at[pl.ds(s * TPR, TPR)], zsem)
    base_row = pl.multiple_of((c * NS + s) * CHW, 8)
    pltpu.sync_copy(src_hbm.at[pl.ds(base_row, IB)], sidx.at[0])
    pltpu.sync_copy(dst_hbm.at[pl.ds(base_row, IB)], didx.at[0])
    for t in range(F):
        pltpu.async_copy(table_hbm.at[sidx.at[0, t]], rows_bf.at[t], gsem)
    zdesc.wait()
    plsc.subcore_barrier()

    def gdrain():
        pltpu.make_async_copy(table_hbm.at[pl.ds(0, K)], rows_bf.at[0],
                              gsem).wait()

    def sdrain():
        pltpu.make_async_copy(out_hbm.at[0, pl.ds(0, K)], rows_f32.at[0],
                              ssem).wait()

    def body(t, _):
        blk = t // IB
        r_in = t % IB
        nxt_ok = blk + 1 < NIBW

        @pl.when(jnp.logical_and(r_in == 0, nxt_ok))
        def _():
            nxt = (blk + 1) % 2
            row0 = pl.multiple_of(base_row + (blk + 1) * IB, 8)
            pltpu.async_copy(src_hbm.at[pl.ds(row0, IB)], sidx.at[nxt], isem)
            pltpu.async_copy(dst_hbm.at[pl.ds(row0, IB)], didx.at[nxt], isem)

        @pl.when(jnp.logical_and(r_in == IB - F - 1, nxt_ok))
        def _():
            for ref in (sidx, didx):
                pltpu.make_async_copy(src_hbm.at[pl.ds(base_row, IB)],
                                      ref.at[0], isem).wait()

        gdrain()
        sb = lax.rem(t, S)

        @pl.when(t >= S)
        def _():
            sdrain()

        b4 = lax.rem(t, R)

        def conv(r, _):
            for g in range(4):
                w = rows_bf[b4, r, pl.ds(g * 16, 16)]
                lo = lax.bitcast_convert_type(w << 16, jnp.float32)
                hi = lax.bitcast_convert_type(w & jnp.int32(-65536),
                                              jnp.float32)
                rows_f32[sb, r, pl.ds(g * 32, 16)] = lo
                rows_f32[sb, r, pl.ds(g * 32 + 16, 16)] = hi
            return 0

        lax.fori_loop(0, K, conv, 0)

        cur = lax.rem(blk, 2)
        pltpu.async_copy(rows_f32.at[sb], agg_sh.at[didx.at[cur, r_in]],
                         ssem, add=True)

        @pl.when(t + F < CHW)
        def _():
            tf = t + F
            pltpu.async_copy(table_hbm.at[sidx.at[lax.rem(tf // IB, 2),
                                                  lax.rem(tf, IB)]],
                             rows_bf.at[lax.rem(tf, R)], gsem)
        return 0

    lax.fori_loop(0, CHW, body, 0)
    for _ in range(S):
        sdrain()
    plsc.subcore_barrier()
    pltpu.sync_copy(agg_sh.at[pl.ds(s * TPR, TPR)],
                    out_hbm.at[c, pl.ds(s * TPR, TPR)])


_sc_agg = functools.partial(
    pl.kernel,
    out_type=jax.ShapeDtypeStruct((NCU, NROWS, D), jnp.float32),
    mesh=plsc.VectorSubcoreMesh(core_axis_name="c", subcore_axis_name="s",
                                num_cores=NCU),
    compiler_params=pltpu.CompilerParams(use_tc_tiling_on_sc=False),
    scratch_types=[
        pltpu.VMEM((2, IB, K), jnp.int32),
        pltpu.VMEM((2, IB, K), jnp.int32),
        pltpu.VMEM((R, K, D // 2), jnp.int32),
        pltpu.VMEM((S, K, D), jnp.float32),
        pltpu.VMEM_SHARED((NROWS, D), jnp.float32),
        pltpu.SemaphoreType.DMA,
        pltpu.SemaphoreType.DMA,
        pltpu.SemaphoreType.DMA,
        pltpu.SemaphoreType.DMA,
    ],
)(_sc_agg_body)


def _tc_layer1_body(p_ref, x_ref, wrel_ref, wroot_ref, b_ref, pm_ref, h_ref):
    agg = p_ref[0] + p_ref[1]
    h = (jnp.dot(agg, wrel_ref[...], preferred_element_type=jnp.float32)
         + jnp.dot(x_ref[...], wroot_ref[...],
                   preferred_element_type=jnp.float32)
         + b_ref[...])
    h = jnp.maximum(h, 0.0)
    hsw = jnp.dot(h, pm_ref[...], preferred_element_type=jnp.float32)
    h_ref[...] = hsw.astype(jnp.bfloat16)


def _tc_layer1(p, x, wrelp, wroot, b2d, pm):
    return pl.pallas_call(
        _tc_layer1_body,
        grid=(NBLK,),
        in_specs=[
            pl.BlockSpec((NCU, RB, D), lambda i: (0, i, 0)),
            pl.BlockSpec((RB, D), lambda i: (i, 0)),
            pl.BlockSpec((D, D), lambda i: (0, 0)),
            pl.BlockSpec((D, D), lambda i: (0, 0)),
            pl.BlockSpec((1, D), lambda i: (0, 0)),
            pl.BlockSpec((D, D), lambda i: (0, 0)),
        ],
        out_specs=pl.BlockSpec((RB, D), lambda i: (i, 0)),
        out_shape=jax.ShapeDtypeStruct((N, D), jnp.bfloat16),
    )(p, x, wrelp, wroot, b2d, pm)


def _tc_layer2_body(p_ref, h_ref, wrel_ref, wroot_ref, b_ref, batch_ref,
                    out_ref, acc, cnt):
    i = pl.program_id(0)
    agg = p_ref[0] + p_ref[1]
    h2 = (jnp.dot(agg, wrel_ref[...], preferred_element_type=jnp.float32)
          + jnp.dot(h_ref[...], wroot_ref[...],
                    preferred_element_type=jnp.float32)
          + b_ref[...])
    h2 = jnp.maximum(h2, 0.0)

    seg = batch_ref[0, 0, :]
    gid = lax.broadcasted_iota(jnp.int32, (RB, G), 1)
    onehot = jnp.where(seg[:, None] == gid, 1.0, 0.0)
    psum = lax.dot_general(onehot, h2, (((0,), (0,)), ((), ())),
                           preferred_element_type=jnp.float32)
    pcnt = jnp.sum(onehot, axis=0)[None, :]

    @pl.when(i == 0)
    def _():
        acc[...] = psum
        cnt[...] = pcnt

    @pl.when(i > 0)
    def _():
        acc[...] += psum
        cnt[...] += pcnt

    @pl.when(i == NBLK - 1)
    def _():
        denom = jnp.maximum(cnt[...], 1.0)
        out_ref[...] = acc[...] / denom[0, :, None]


def _tc_layer2(p, h, wrelp, wrootp, b2d, batch3):
    return pl.pallas_call(
        _tc_layer2_body,
        grid=(NBLK,),
        in_specs=[
            pl.BlockSpec((NCU, RB, D), lambda i: (0, i, 0)),
            pl.BlockSpec((RB, D), lambda i: (i, 0)),
            pl.BlockSpec((D, D), lambda i: (0, 0)),
            pl.BlockSpec((D, D), lambda i: (0, 0)),
            pl.BlockSpec((1, D), lambda i: (0, 0)),
            pl.BlockSpec((1, 1, RB), lambda i: (i, 0, 0)),
        ],
        out_specs=pl.BlockSpec((G, D), lambda i: (0, 0)),
        out_shape=jax.ShapeDtypeStruct((G, D), jnp.float32),
        scratch_shapes=[
            pltpu.VMEM((G, D), jnp.float32),
            pltpu.VMEM((1, G), jnp.float32),
        ],
        compiler_params=pltpu.CompilerParams(
            dimension_semantics=("arbitrary",),
        ),
    )(p, h, wrelp, wrootp, b2d, batch3)


def _pack_i32(a_bf16):
    return lax.bitcast_convert_type(
        a_bf16.reshape(a_bf16.shape[0], D // 2, 2), jnp.int32)


def kernel(x, edge_index, batch, W_rel1, b_rel1, W_root1, W_rel2, b_rel2,
           W_root2):
    cm = jnp.asarray(_CM)
    pm = jnp.asarray(_PM)
    src = edge_index[0]
    dst = edge_index[1]
    npad = EPAD - E
    src_p = jnp.concatenate([src, jnp.zeros((npad,), jnp.int32)])
    src_p = src_p.reshape(EPAD // K, K)
    trash = N + (jnp.arange(npad, dtype=jnp.int32) % (NROWS - N))
    dst_p = jnp.concatenate([dst, trash]).reshape(EPAD // K, K)
    zrows = jnp.zeros((TPR, D), jnp.float32)
    b1 = b_rel1.reshape(1, D)
    b2 = b_rel2.reshape(1, D)
    batch3 = batch.reshape(NBLK, 1, RB)

    wrel1p = W_rel1[cm, :]
    wrel2p = W_rel2[cm, :]
    wroot2p = W_root2[cm, :].astype(jnp.bfloat16)

    xsw = _pack_i32(x[:, cm].astype(jnp.bfloat16))
    p1 = _sc_agg(xsw, src_p, dst_p, zrows)
    hsw = _tc_layer1(p1, x, wrel1p, W_root1, b1, pm)
    p2 = _sc_agg(_pack_i32(hsw), src_p, dst_p, zrows)
    out = _tc_layer2(p2, hsw, wrel2p, wroot2p, b2, batch3)
    return out
